# baseline (speedup 1.0000x reference)
import functools
import jax
import jax.numpy as jnp
from jax.experimental import pallas as pl

N_SUB = 10000
N_REG = 50000
H = 128
OUT = 64
EPS = 1e-5

_ROWS_BLK = 512


def _mm_kernel(x_ref, w_ref, b_ref, o_ref):
    o_ref[...] = (
        jnp.dot(x_ref[...], w_ref[...], preferred_element_type=jnp.float32)
        + b_ref[...]
    )


def _mm_bias(x, w, b):
    n, k = x.shape
    ko, m = w.shape
    pad = (-n) % _ROWS_BLK
    xp = jnp.pad(x, ((0, pad), (0, 0))) if pad else x
    np_ = n + pad
    out = pl.pallas_call(
        _mm_kernel,
        grid=(np_ // _ROWS_BLK,),
        in_specs=[
            pl.BlockSpec((_ROWS_BLK, k), lambda i: (i, 0)),
            pl.BlockSpec((k, m), lambda i: (0, 0)),
            pl.BlockSpec((1, m), lambda i: (0, 0)),
        ],
        out_specs=pl.BlockSpec((_ROWS_BLK, m), lambda i: (i, 0)),
        out_shape=jax.ShapeDtypeStruct((np_, m), jnp.float32),
    )(xp, w, b.reshape(1, m))
    return out[:n] if pad else out


def _bn_relu(x, gamma, beta):
    scale = gamma / jnp.sqrt(1.0 + EPS)
    return jax.nn.relu(x * scale + beta)


def _graph_conv(x_src, x_dst, src, dst, Wrel, brel, Wroot, n_dst):
    agg = jax.ops.segment_sum(x_src[src], dst, num_segments=n_dst)
    return _mm_bias(agg, Wrel, brel) + _mm_bias(x_dst, Wroot, jnp.zeros((Wroot.shape[1],), jnp.float32))


def _gat_conv(x, src, dst, W, a_src, a_dst, b, n_dst):
    g = _mm_bias(x, W, jnp.zeros((W.shape[1],), jnp.float32))
    es = g @ a_src
    ed = g @ a_dst
    loop = jnp.arange(n_dst, dtype=src.dtype)
    s = jnp.concatenate([src, loop])
    d = jnp.concatenate([dst, loop])
    e = jax.nn.leaky_relu(es[s] + ed[d], 0.2)
    emax = jax.ops.segment_max(e, d, num_segments=n_dst)
    ex = jnp.exp(e - emax[d])
    denom = jax.ops.segment_sum(ex, d, num_segments=n_dst)
    alpha = ex / (denom[d] + 1e-16)
    out = jax.ops.segment_sum(alpha[:, None] * g[s], d, num_segments=n_dst)
    return out + b


def kernel(x_subject, x_region, params, ei_hr_src, ei_hr_dst, ei_rev_src, ei_rev_dst, ei_rr_src, ei_rr_dst):
    p = params
    h_sub = _mm_bias(x_subject, p['W_sub'], p['b_sub'])
    h_reg = _mm_bias(x_region, p['W_reg'], p['b_reg'])
    for lp in p['layers']:
        reg_in = _graph_conv(h_sub, h_reg, ei_hr_src, ei_hr_dst, lp['hr_Wrel'], lp['hr_brel'], lp['hr_Wroot'], N_REG)
        sub_in = _graph_conv(h_reg, h_sub, ei_rev_src, ei_rev_dst, lp['rev_Wrel'], lp['rev_brel'], lp['rev_Wroot'], N_SUB)
        reg_gat = _gat_conv(h_reg, ei_rr_src, ei_rr_dst, lp['gat_W'], lp['gat_asrc'], lp['gat_adst'], lp['gat_b'], N_REG)
        h_sub = _bn_relu(sub_in, lp['bn_gamma'], lp['bn_beta'])
        h_reg = _bn_relu(reg_in + reg_gat, lp['bn_gamma'], lp['bn_beta'])
    out_sub = _mm_bias(h_sub, p['Wo_sub'], p['bo_sub'])
    out_reg = _mm_bias(h_reg, p['Wo_reg'], p['bo_reg'])
    return out_sub, out_reg
